# Initial kernel scaffold; baseline (speedup 1.0000x reference)
import jax
import jax.numpy as jnp
from jax.experimental import pallas as pl


def kernel(x0, x1, r12_w3, r12_b3, r12_w1, r12_b1, r34_w3, r34_b3, r34_w1, r34_b1, ct1_w, ct1_b, ct3_w, ct3_b, ct4_w, ct4_b, c2_wa, c2_wb, c2_b):
    raise NotImplementedError("write your pallas kernel here")


# speedup vs baseline: 1.1286x; 1.1286x over previous
import jax
import jax.numpy as jnp
from jax.experimental import pallas as pl
from jax.experimental.pallas import tpu as pltpu

_VMEM_LIMIT = 40 * 1024 * 1024


def _cparams():
    return pltpu.CompilerParams(
        dimension_semantics=("parallel",),
        vmem_limit_bytes=_VMEM_LIMIT,
    )


def _im2col9(x, H, W, C):
    cols = [x[:, kx:kx + W, :] for kx in range(3)]
    return jnp.concatenate(
        [cols[kx][ky:ky + H].reshape(H * W, C)
         for kx in range(3) for ky in range(3)],
        axis=-1)


def _halo_store(o_ref, interior, H, W, C):
    o_ref[0] = jnp.zeros((H + 2, W + 2, C), o_ref.dtype)
    o_ref[0, 1:1 + H, 1:1 + W, :] = interior.astype(o_ref.dtype)


def _res_pair_body(H, W, C):
    HW = H * W

    def body(xp_ref, w9_ref, b3_ref, w1_ref, b1_ref, o_ref, mid_ref):
        def one(xp, blk):
            xr = jnp.maximum(xp, 0).astype(jnp.bfloat16)
            x9 = _im2col9(xr, H, W, C)
            acc = jnp.dot(x9, w9_ref[blk], preferred_element_type=jnp.float32)
            h = jnp.maximum(acc + b3_ref[blk], 0.0).astype(jnp.bfloat16)
            out = jnp.dot(h, w1_ref[blk],
                          preferred_element_type=jnp.float32) + b1_ref[blk]
            skip = xp[1:1 + H, 1:1 + W, :].reshape(HW, C).astype(jnp.float32)
            return (out + skip).reshape(H, W, C)

        x1 = one(xp_ref[0], 0)
        mid_ref[...] = jnp.zeros((H + 2, W + 2, C), jnp.bfloat16)
        mid_ref[1:1 + H, 1:1 + W, :] = x1.astype(jnp.bfloat16)
        x2 = one(mid_ref[...], 1)
        _halo_store(o_ref, x2, H, W, C)

    return body


def _res_pair(xp, w9, b3, w1, b1):
    B, Hp, Wp, C = xp.shape
    H, W = Hp - 2, Wp - 2
    return pl.pallas_call(
        _res_pair_body(H, W, C),
        out_shape=jax.ShapeDtypeStruct((B, Hp, Wp, C), xp.dtype),
        grid=(B,),
        in_specs=[
            pl.BlockSpec((1, Hp, Wp, C), lambda b: (b, 0, 0, 0)),
            pl.BlockSpec((2, 9 * C, C), lambda b: (0, 0, 0)),
            pl.BlockSpec((2, 1, C), lambda b: (0, 0, 0)),
            pl.BlockSpec((2, C, C), lambda b: (0, 0, 0)),
            pl.BlockSpec((2, 1, C), lambda b: (0, 0, 0)),
        ],
        out_specs=pl.BlockSpec((1, Hp, Wp, C), lambda b: (b, 0, 0, 0)),
        scratch_shapes=[pltpu.VMEM((Hp, Wp, C), jnp.bfloat16)],
        compiler_params=_cparams(),
    )(xp, w9, b3, w1, b1)


def _cat_conv_body(H, W, C):
    HW = H * W

    def body(ap_ref, bp_ref, w_ref, bias_ref, o_ref):
        a = ap_ref[0]
        b = bp_ref[0]
        acols = [a[:, kx:kx + W, :] for kx in range(3)]
        bcols = [b[:, kx:kx + W, :] for kx in range(3)]
        pieces = []
        for kx in range(3):
            for ky in range(3):
                pieces.append(acols[kx][ky:ky + H].reshape(HW, C))
                pieces.append(bcols[kx][ky:ky + H].reshape(HW, C))
        x18 = jnp.concatenate(pieces, axis=-1)
        out = jnp.dot(x18, w_ref[...],
                      preferred_element_type=jnp.float32) + bias_ref[...]
        _halo_store(o_ref, out.reshape(H, W, C), H, W, C)

    return body


def _cat_conv(ap, bp, w, bias):
    B, Hp, Wp, C = ap.shape
    H, W = Hp - 2, Wp - 2
    return pl.pallas_call(
        _cat_conv_body(H, W, C),
        out_shape=jax.ShapeDtypeStruct((B, Hp, Wp, C), ap.dtype),
        grid=(B,),
        in_specs=[
            pl.BlockSpec((1, Hp, Wp, C), lambda b: (b, 0, 0, 0)),
            pl.BlockSpec((1, Hp, Wp, C), lambda b: (b, 0, 0, 0)),
            pl.BlockSpec((18 * C, C), lambda b: (0, 0)),
            pl.BlockSpec((1, C), lambda b: (0, 0)),
        ],
        out_specs=pl.BlockSpec((1, Hp, Wp, C), lambda b: (b, 0, 0, 0)),
        compiler_params=_cparams(),
    )(ap, bp, w, bias)


def _convt_body(H, W, C, Cop, Cst, relu_in, relu_out):
    HW = H * W

    def body(xp_ref, w_ref, b_ref, o_ref):
        x = xp_ref[0]
        if relu_in:
            x = jnp.maximum(x, 0)
        x = x.astype(jnp.bfloat16)
        cols = [x[:, c:c + W, :] for c in range(3)]
        s = [[cols[c][r:r + H].reshape(HW, C) for r in range(3)]
             for c in range(3)]
        bias = b_ref[...]
        for py in range(2):
            lhs = jnp.concatenate(
                [s[c][py + dy] for c in range(3) for dy in range(2)], axis=-1)
            acc = jnp.dot(lhs, w_ref[py],
                          preferred_element_type=jnp.float32) + bias
            if relu_out:
                acc = jnp.maximum(acc, 0.0)
            o_ref[2 * py] = acc[:, :Cst].reshape(H, W, Cst).astype(o_ref.dtype)
            o_ref[2 * py + 1] = acc[:, Cop:Cop + Cst].reshape(
                H, W, Cst).astype(o_ref.dtype)

    return body


def _convt_up2(xp, wpk, bias2, *, relu_in, relu_out, store_ch=None,
               out_dtype=None):
    B, Hp, Wp, C = xp.shape
    H, W = Hp - 2, Wp - 2
    Cop = wpk.shape[-1] // 2
    Cst = Cop if store_ch is None else store_ch
    out_dtype = xp.dtype if out_dtype is None else out_dtype
    return pl.pallas_call(
        _convt_body(H, W, C, Cop, Cst, relu_in, relu_out),
        out_shape=jax.ShapeDtypeStruct((B * 4, H, W, Cst), out_dtype),
        grid=(B,),
        in_specs=[
            pl.BlockSpec((1, Hp, Wp, C), lambda b: (b, 0, 0, 0)),
            pl.BlockSpec((2, 6 * C, 2 * Cop), lambda b: (0, 0, 0)),
            pl.BlockSpec((1, 2 * Cop), lambda b: (0, 0)),
        ],
        out_specs=pl.BlockSpec((4, H, W, Cst), lambda b: (b, 0, 0, 0)),
        compiler_params=_cparams(),
    )(xp, wpk, bias2)


def _d2s_pad(planar, B):
    _, H, W, C = planar.shape
    y = planar.reshape(B, 2, 2, H, W, C)
    y = jnp.transpose(y, (0, 3, 1, 4, 2, 5)).reshape(B, 2 * H, 2 * W, C)
    return jnp.pad(y, ((0, 0), (1, 1), (1, 1), (0, 0)))


def _d2s_nchw(planar, B):
    _, H, W, C = planar.shape
    y = planar.reshape(B, 2, 2, H, W, C)
    return jnp.transpose(y, (0, 5, 3, 1, 4, 2)).reshape(B, C, 2 * H, 2 * W)


def _nchw_to_padded_nhwc(x_nchw):
    x = jnp.transpose(x_nchw, (0, 2, 3, 1))
    x = jnp.pad(x, ((0, 0), (1, 1), (1, 1), (0, 0)))
    return x.astype(jnp.bfloat16)


def _pack_w9(w3):
    return jnp.concatenate(
        [w3[:, ky * 3 + kx] for kx in range(3) for ky in range(3)], axis=1)


def _pack_cat_w(wa, wb):
    parts = []
    for kx in range(3):
        for ky in range(3):
            t = ky * 3 + kx
            parts.append(wa[t])
            parts.append(wb[t])
    return jnp.concatenate(parts, axis=0)


def _pack_ct_w(wpar):
    C, Cop = wpar.shape[-2], wpar.shape[-1]
    z = jnp.zeros((C, Cop), wpar.dtype)
    rows = []
    for py in range(2):
        kblocks = []
        for c in range(3):
            for dy in range(2):
                left = wpar[2 * py, 2 * dy + c] if c <= 1 else z
                right = wpar[2 * py + 1, 2 * dy + c - 1] if c >= 1 else z
                kblocks.append(jnp.concatenate([left, right], axis=1))
        rows.append(jnp.concatenate(kblocks, axis=0))
    return jnp.stack(rows)


def _pack_ct_b(b):
    return jnp.concatenate([b, b], axis=1)


def kernel(x0, x1, r12_w3, r12_b3, r12_w1, r12_b1,
           r34_w3, r34_b3, r34_w1, r34_b1,
           ct1_w, ct1_b, ct3_w, ct3_b, ct4_w, ct4_b,
           c2_wa, c2_wb, c2_b):
    B = x0.shape[0]
    xp = _nchw_to_padded_nhwc(x0)
    yp = _nchw_to_padded_nhwc(x1)

    xp = _res_pair(xp, _pack_w9(r12_w3), r12_b3, r12_w1, r12_b1)
    a = _convt_up2(xp, _pack_ct_w(ct1_w), _pack_ct_b(ct1_b),
                   relu_in=True, relu_out=True)
    ap = _d2s_pad(a, B)
    xp = _cat_conv(ap, yp, _pack_cat_w(c2_wa, c2_wb), c2_b)
    xp = _res_pair(xp, _pack_w9(r34_w3), r34_b3, r34_w1, r34_b1)
    a = _convt_up2(xp, _pack_ct_w(ct3_w), _pack_ct_b(ct3_b),
                   relu_in=True, relu_out=True)
    xp = _d2s_pad(a, B)
    out = _convt_up2(xp, _pack_ct_w(ct4_w), _pack_ct_b(ct4_b),
                     relu_in=False, relu_out=False,
                     store_ch=3, out_dtype=jnp.float32)
    return _d2s_nchw(out, B)
